# baseline (speedup 1.0000x reference)
import functools

import jax
import jax.numpy as jnp
from jax import lax
from jax.experimental import pallas as pl
from jax.experimental.pallas import tpu as pltpu
from jax.experimental.pallas import tpu_sc as plsc

N = 10000
E = 160000
D = 256
S = 2
NC = 2
NS = 16
NW = NC * NS
DC = 128
NJ = D // DC
CB = 128
NK = 40
CB2 = 32
NK2 = 160
NKC = NK2
EPW = NK * CB
EPAD = NW * EPW
NR = 2 * NJ * S
SR = 632
SRL = N - 15 * SR
BN = 400
NT = N // BN


def _t0_body(x_ref, wl_ref, sl_ref, g_ref):
    for s2 in range(2):
        sl_ref[s2] = jnp.dot(x_ref[s2], wl_ref[...],
                             preferred_element_type=jnp.float32)
    for j in range(NJ):
        g_ref[j] = x_ref[1, :, j * DC:(j + 1) * DC]
        g_ref[NJ + j] = x_ref[0, :, j * DC:(j + 1) * DC]


def _t0_call(x, w_loops):
    return pl.pallas_call(
        _t0_body,
        grid=(NT,),
        in_specs=[
            pl.BlockSpec((2, BN, D), lambda t: (0, t, 0)),
            pl.BlockSpec((D, D), lambda t: (0, 0)),
        ],
        out_specs=[
            pl.BlockSpec((2, BN, D), lambda t: (0, t, 0)),
            pl.BlockSpec((2 * NJ, BN, DC), lambda t: (0, t, 0)),
        ],
        out_shape=[
            jax.ShapeDtypeStruct((2, N, D), jnp.float32),
            jax.ShapeDtypeStruct((2 * NJ, N, DC), jnp.float32),
        ],
    )(x, w_loops)


def _transform_common(sl_ref, p_ref, w_ref, h_ref, hc_ref):
    hs = []
    for s2 in range(2):
        acc = sl_ref[s2]
        for i in range(S):
            a = None
            for c in range(NC):
                cat = jnp.concatenate(
                    [p_ref[s2, j, i, c, :, :] for j in range(NJ)], axis=-1)
                a = cat if a is None else a + cat
            acc = acc + jnp.dot(a, w_ref[i], preferred_element_type=jnp.float32)
        h = jnp.maximum(acc, 0.0)
        h_ref[s2] = h
        hs.append(h)
    if hc_ref is not None:
        for j in range(NJ):
            hc_ref[j] = hs[1][:, j * DC:(j + 1) * DC]
            hc_ref[NJ + j] = hs[0][:, j * DC:(j + 1) * DC]


def _transform_body_l0(sl_ref, p_ref, w_ref, h_ref, hc_ref):
    _transform_common(sl_ref, p_ref, w_ref, h_ref, hc_ref)


def _transform_body_l1(sl_ref, p_ref, w_ref, h_ref):
    _transform_common(sl_ref, p_ref, w_ref, h_ref, None)


def _transform_call(sl, p, wl, want_chunks):
    out_shape = [jax.ShapeDtypeStruct((2, N, D), jnp.float32)]
    out_specs = [pl.BlockSpec((2, BN, D), lambda t: (0, t, 0))]
    if want_chunks:
        out_shape.append(jax.ShapeDtypeStruct((2 * NJ, N, DC), jnp.float32))
        out_specs.append(pl.BlockSpec((2 * NJ, BN, DC), lambda t: (0, t, 0)))
    return pl.pallas_call(
        _transform_body_l0 if want_chunks else _transform_body_l1,
        grid=(NT,),
        in_specs=[
            pl.BlockSpec((2, BN, D), lambda t: (0, t, 0)),
            pl.BlockSpec((2, NJ, S, NC, BN, DC),
                         lambda t: (0, 0, 0, 0, t, 0)),
            pl.BlockSpec((S, D, D), lambda t: (0, 0, 0)),
        ],
        out_specs=out_specs,
        out_shape=out_shape,
    )(sl, p, wl)


_sc_mesh = plsc.VectorSubcoreMesh(core_axis_name="c", subcore_axis_name="s")


@functools.partial(
    pl.kernel,
    out_type=[
        jax.ShapeDtypeStruct((2, NJ, S, NC, N, DC), jnp.float32),
        jax.ShapeDtypeStruct((NW, NK2, CB2, DC), jnp.float32),
    ],
    mesh=_sc_mesh,
    scratch_types=[
        pltpu.VMEM((NK, CB), jnp.int32),
        pltpu.VMEM((NK, CB), jnp.int32),
        pltpu.VMEM((NK, CB), jnp.float32),
        pltpu.VMEM((2, CB2, DC), jnp.float32),
        pltpu.VMEM((2, CB2, DC), jnp.float32),
        pltpu.VMEM_SHARED((N, DC), jnp.float32),
        pltpu.SemaphoreType.DMA((8,)),
    ],
)
def _sc_spmm(g_hbm,
             dst_hbm, src_hbm, vals_hbm, zeros_hbm, out_hbm, raw_hbm,
             dst_v, src_v, vbuf, gbuf, sbuf, acc, sems):
    core = lax.axis_index("c")
    sid = lax.axis_index("s")
    wid = core * NS + sid
    gsems = (sems.at[0], sems.at[1])
    ssems = (sems.at[2], sems.at[3])
    rwsems = (sems.at[4], sems.at[5])
    zsem = sems.at[6]
    base = sid * SR

    def idx16(buf, c, g):
        return buf[c // 4, pl.ds((c % 4) * CB2 + g * 16, 16)]

    def zero_acc():
        @pl.when(sid < 15)
        def _():
            pltpu.async_copy(zeros_hbm, acc.at[pl.ds(base, SR)], zsem)
            pltpu.make_async_copy(
                zeros_hbm, acc.at[pl.ds(base, SR)], zsem).wait()

        @pl.when(sid == 15)
        def _():
            pltpu.async_copy(
                zeros_hbm.at[pl.ds(0, SRL)], acc.at[pl.ds(base, SRL)], zsem)
            pltpu.make_async_copy(
                zeros_hbm.at[pl.ds(0, SRL)], acc.at[pl.ds(base, SRL)],
                zsem).wait()

    def scale_chunk(b, c):
        def group_body(g, c3):
            vv = idx16(vbuf, c, g)
            for t in range(16):
                e = g * 16 + t
                v = jnp.full((16,), vv[t], jnp.float32)

                def ddb(dd, c4):
                    x = gbuf[b, e, pl.ds(dd * 16, 16)]
                    sbuf[b, e, pl.ds(dd * 16, 16)] = x * v
                    return c4
                lax.fori_loop(0, DC // 16, ddb, 0)
            return c3
        lax.fori_loop(0, CB2 // 16, group_body, 0)

    def fire_scatters(b, c):
        for g in range(CB2 // 16):
            pltpu.async_copy(
                sbuf.at[b, pl.ds(g * 16, 16)],
                acc.at[idx16(dst_v, c, g)], ssems[b], add=True)

    def wait_scatters(b, c):
        for g in range(CB2 // 16):
            pltpu.make_async_copy(
                sbuf.at[b, pl.ds(g * 16, 16)],
                acc.at[idx16(dst_v, c, g)], ssems[b]).wait()

    def drain_out(side, jj, sup):
        @pl.when(sid < 15)
        def _():
            pltpu.sync_copy(
                acc.at[pl.ds(base, SR)],
                out_hbm.at[side, jj, sup, core, pl.ds(base, SR)])

        @pl.when(sid == 15)
        def _():
            pltpu.sync_copy(
                acc.at[pl.ds(15 * SR, SRL)],
                out_hbm.at[side, jj, sup, core, pl.ds(15 * SR, SRL)])

    def round_pair(r2, carry):
        side = r2 // NJ
        jj = r2 % NJ
        gsub2 = g_hbm.at[r2]
        pltpu.sync_copy(dst_hbm.at[side, wid], dst_v)
        pltpu.sync_copy(src_hbm.at[side, wid], src_v)

        pltpu.sync_copy(vals_hbm.at[side, 0, wid], vbuf)
        zero_acc()
        plsc.subcore_barrier()

        def fire_gathers(c, b):
            for g in range(CB2 // 16):
                pltpu.async_copy(
                    gsub2.at[idx16(src_v, c, g)],
                    gbuf.at[b, pl.ds(g * 16, 16)], gsems[b])

        def wait_gathers(c, b):
            for g in range(CB2 // 16):
                pltpu.make_async_copy(
                    gsub2.at[idx16(src_v, c, g)],
                    gbuf.at[b, pl.ds(g * 16, 16)], gsems[b]).wait()

        fire_gathers(0, 0)

        def chunk_pair(kk, c2):
            for b in range(2):
                c = kk * 2 + b
                bo = 1 - b
                wait_gathers(c, b)
                pltpu.async_copy(
                    gbuf.at[b], raw_hbm.at[wid, c], rwsems[b])

                @pl.when(c >= 1)
                def _():
                    pltpu.make_async_copy(
                        gbuf.at[bo], raw_hbm.at[wid, c - 1],
                        rwsems[bo]).wait()
                    wait_scatters(bo, c - 1)

                @pl.when(c + 1 < NKC)
                def _():
                    fire_gathers(c + 1, bo)
                scale_chunk(b, c)
                fire_scatters(b, c)
            return c2
        lax.fori_loop(0, NKC // 2, chunk_pair, 0)
        pltpu.make_async_copy(
            gbuf.at[1], raw_hbm.at[wid, NKC - 1], rwsems[1]).wait()
        wait_scatters(1, NKC - 1)
        plsc.subcore_barrier()
        drain_out(side, jj, 0)

        pltpu.sync_copy(vals_hbm.at[side, 1, wid], vbuf)
        zero_acc()
        plsc.subcore_barrier()
        pltpu.async_copy(raw_hbm.at[wid, 0], gbuf.at[0], gsems[0])

        def chunk_pair1(kk, c2):
            for b in range(2):
                c = kk * 2 + b
                bo = 1 - b
                pltpu.make_async_copy(
                    raw_hbm.at[wid, c], gbuf.at[b], gsems[b]).wait()

                @pl.when(c >= 1)
                def _():
                    wait_scatters(bo, c - 1)

                @pl.when(c + 1 < NKC)
                def _():
                    pltpu.async_copy(
                        raw_hbm.at[wid, c + 1], gbuf.at[bo], gsems[bo])
                scale_chunk(b, c)
                fire_scatters(b, c)
            return c2
        lax.fori_loop(0, NKC // 2, chunk_pair1, 0)
        wait_scatters(1, NKC - 1)
        plsc.subcore_barrier()
        drain_out(side, jj, 1)
        return carry
    lax.fori_loop(0, 2 * NJ, round_pair, 0)


def _pad_edges(edges, vals):
    pad = EPAD - E
    dst = jnp.concatenate([edges[0], jnp.zeros((pad,), jnp.int32)])
    src = jnp.concatenate([edges[1], jnp.zeros((pad,), jnp.int32)])
    v = jnp.concatenate([vals, jnp.zeros((S, pad), jnp.float32)], axis=1)
    return (dst.reshape(NW, NK, CB), src.reshape(NW, NK, CB),
            v.reshape(S, NW, NK, CB))


def kernel(rna_edges, prot_edges, rna_vals, prot_vals,
           RNA_inputs, protein_inputs, W0, W1, W_loops):
    x = jnp.stack([RNA_inputs, protein_inputs])
    dst_r, src_r, v_r = _pad_edges(rna_edges, rna_vals)
    dst_p, src_p, v_p = _pad_edges(prot_edges, prot_vals)
    dst = jnp.stack([dst_r, dst_p])
    src = jnp.stack([src_r, src_p])
    vals = jnp.stack([v_r, v_p])

    zeros = jnp.zeros((SR, DC), jnp.float32)
    sl, g = _t0_call(x, W_loops)
    h = None
    for l, wl in enumerate((W0, W1)):
        p, _ = _sc_spmm(g, dst, src, vals, zeros)
        if l == 0:
            h, g = _transform_call(sl, p, wl, want_chunks=True)
        else:
            h = _transform_call(sl, p, wl, want_chunks=False)[0]
    return (h[0], h[1])
